# baseline (speedup 1.0000x reference)
import functools

import jax
import jax.numpy as jnp
from jax import lax
from jax.experimental import pallas as pl
from jax.experimental.pallas import tpu as pltpu
from jax.experimental.pallas import tpu_sc as plsc

_N = 4096 * 200
_D = 64
_DP = 2 * _D
_NW = 32

_K = 17
_CP = 256
_CH = _CP // 2
_SC_PAIRS = _K * _CP * _NW
_PAIRS_W = _K * _CP
_NCH = _K

_TC_ROWS = _N - 2 * _SC_PAIRS
_TC_BR = 8192
_TC_BLOCKS = _TC_ROWS // _TC_BR

_mesh = plsc.VectorSubcoreMesh(core_axis_name="c", subcore_axis_name="s")


@functools.partial(
    pl.kernel,
    mesh=_mesh,
    out_type=jax.ShapeDtypeStruct((_SC_PAIRS, _DP), jnp.float32),
    scratch_types=[
        pltpu.VMEM_SHARED((1024, _DP), jnp.float32),
        pltpu.VMEM((_CH,), jnp.int32),
        pltpu.VMEM((_CH,), jnp.int32),
        pltpu.VMEM((_CH,), jnp.int32),
        pltpu.VMEM((_CH,), jnp.int32),
        pltpu.VMEM((_CH,), jnp.int32),
        pltpu.VMEM((_CH,), jnp.int32),
        pltpu.VMEM((_CP, _DP), jnp.float32),
        pltpu.VMEM((_CP, _DP), jnp.float32),
        pltpu.VMEM((_CP, _DP), jnp.float32),
        pltpu.SemaphoreType.DMA,
        pltpu.SemaphoreType.DMA,
        pltpu.SemaphoreType.DMA,
        pltpu.SemaphoreType.DMA,
        pltpu.SemaphoreType.DMA,
        pltpu.SemaphoreType.DMA,
        pltpu.SemaphoreType.DMA,
        pltpu.SemaphoreType.DMA,
        pltpu.SemaphoreType.DMA,
        pltpu.SemaphoreType.DMA,
        pltpu.SemaphoreType.DMA,
    ],
)
def _sc_body(idx_hbm, x_hbm, table2_hbm, out_hbm, table_sh,
             idx_va0, idx_va1, idx_va2, idx_vb0, idx_vb1, idx_vb2,
             x_v0, x_v1, x_v2,
             si0, si1, si2, sx0, sx1, sx2, so0, so1, so2, sg, sg2):
    sid = lax.axis_index("s")
    wid = sid * 2 + lax.axis_index("c")
    pbase = wid * _PAIRS_W
    idx_va = (idx_va0, idx_va1, idx_va2)
    idx_vb = (idx_vb0, idx_vb1, idx_vb2)
    x_v = (x_v0, x_v1, x_v2)
    si = (si0, si1, si2)
    sx = (sx0, sx1, sx2)
    so = (so0, so1, so2)

    @pl.when(sid == 0)
    def _load_table():
        pltpu.sync_copy(table2_hbm, table_sh)

    plsc.subcore_barrier()

    def load_idx(q, m):
        pltpu.async_copy(idx_hbm.at[pl.ds(q, _CH)], idx_va[m], si[m])
        pltpu.async_copy(idx_hbm.at[pl.ds(q + _CH, _CH)], idx_vb[m], si[m])

    def wait_idx(q, m):
        pltpu.make_async_copy(
            idx_hbm.at[pl.ds(q, _CH)], idx_va[m], si[m]).wait()
        pltpu.make_async_copy(
            idx_hbm.at[pl.ds(q + _CH, _CH)], idx_vb[m], si[m]).wait()

    for j in range(3):
        q = pbase + j * _CP
        load_idx(q, j)
        pltpu.async_copy(x_hbm.at[pl.ds(q, _CP)], x_v[j], sx[j])

    def chunk(h3, carry):
        for m in range(3):
            h = 3 * h3 + m

            @pl.when(h < _NCH)
            def _do():
                q0 = pbase + h * _CP
                wait_idx(q0, m)
                pltpu.make_async_copy(
                    x_hbm.at[pl.ds(q0, _CP)], x_v[m], sx[m]).wait()
                pltpu.async_copy(
                    table_sh.at[idx_va[m]],
                    x_v[m].at[pl.ds(0, _CH)], sg, add=True)
                pltpu.async_copy(
                    table_sh.at[idx_vb[m]],
                    x_v[m].at[pl.ds(_CH, _CH)], sg2, add=True)
                pltpu.make_async_copy(
                    table_sh.at[idx_va[m]],
                    x_v[m].at[pl.ds(0, _CH)], sg).wait()
                pltpu.make_async_copy(
                    table_sh.at[idx_vb[m]],
                    x_v[m].at[pl.ds(_CH, _CH)], sg2).wait()
                pltpu.async_copy(
                    x_v[m], out_hbm.at[pl.ds(q0, _CP)], so[m])

                @pl.when(h + 3 < _NCH)
                def _prefetch_idx():
                    load_idx(q0 + 3 * _CP, m)

                mp = (m + 2) % 3
                h2 = h + 2

                @pl.when((h2 >= 3) & (h2 < _NCH))
                def _prefetch_x():
                    q2 = pbase + h2 * _CP
                    pltpu.make_async_copy(
                        x_v[mp], out_hbm.at[pl.ds(q2 - 3 * _CP, _CP)],
                        so[mp]).wait()
                    pltpu.async_copy(
                        x_hbm.at[pl.ds(q2, _CP)], x_v[mp], sx[mp])

        return carry

    lax.fori_loop(0, (_NCH + 2) // 3, chunk, 0)

    for m in range(3):
        pltpu.make_async_copy(
            x_v[m], out_hbm.at[pl.ds(pbase, _CP)], so[m]).wait()


def _tc_body(idx_ref, x_ref, table_ref, out_ref):
    idx = idx_ref[0, 0, :]
    iota = lax.broadcasted_iota(jnp.int32, (32, idx.shape[0]), 0)
    oht = (idx[None, :] == iota).astype(jnp.float32)
    emb = lax.dot_general(
        oht, table_ref[...], (((0,), (0,)), ((), ())),
        preferred_element_type=jnp.float32,
    )
    out_ref[...] = x_ref[...] + emb


def kernel(inputs, x, table):
    B, S, D = x.shape
    idx1 = inputs.reshape(B * S)

    idx_tc = idx1[:_TC_ROWS].reshape(_TC_BLOCKS, 1, _TC_BR)
    x_tc = x.reshape(B * S, D)[:_TC_ROWS]
    out_tc = pl.pallas_call(
        _tc_body,
        grid=(_TC_BLOCKS,),
        in_specs=[
            pl.BlockSpec((1, 1, _TC_BR), lambda i: (i, 0, 0)),
            pl.BlockSpec((_TC_BR, D), lambda i: (i, 0)),
            pl.BlockSpec((32, D), lambda i: (0, 0)),
        ],
        out_specs=pl.BlockSpec((_TC_BR, D), lambda i: (i, 0)),
        out_shape=jax.ShapeDtypeStruct((_TC_ROWS, D), x.dtype),
    )(idx_tc, x_tc, table)

    idx_sc = idx1[_TC_ROWS:]
    idx2 = idx_sc[0::2] * 32 + idx_sc[1::2]
    x_sc = x.reshape(B * S * D)[_TC_ROWS * D:].reshape(_SC_PAIRS, _DP)
    table2 = jnp.concatenate(
        [jnp.repeat(table, 32, axis=0), jnp.tile(table, (32, 1))], axis=1)
    out_sc = _sc_body(idx2, x_sc, table2)

    out = jnp.concatenate(
        [out_tc.reshape(_TC_ROWS * D), out_sc.reshape(_SC_PAIRS * _DP)])
    return out.reshape(B, S, D)


# speedup vs baseline: 1.5027x; 1.5027x over previous
import functools
import math

import jax
import jax.numpy as jnp
from jax import lax
from jax.experimental import pallas as pl
from jax.experimental.pallas import tpu as pltpu
from jax.experimental.pallas import tpu_sc as plsc

_N = 4096 * 200
_D = 64
_N2 = _N // 2
_DP = 2 * _D
_NW = 32

_CP = 256
_CH = _CP // 2
_PAIRS_W = _N2 // _NW
_NCH = _PAIRS_W // _CP

_sc_mesh = plsc.VectorSubcoreMesh(core_axis_name="c", subcore_axis_name="s")


def _sc_fn(idx2_hbm, x_hbm, table2_hbm, out_hbm, table_sh):

    def inner(idx_va0, idx_va1, idx_va2, idx_vb0, idx_vb1, idx_vb2,
              x_v0, x_v1, x_v2,
              si0, si1, si2, sx0, sx1, sx2, so0, so1, so2, sg, sg2):
        sid = lax.axis_index("s")
        wid = sid * 2 + lax.axis_index("c")
        pbase = wid * _PAIRS_W
        idx_va = (idx_va0, idx_va1, idx_va2)
        idx_vb = (idx_vb0, idx_vb1, idx_vb2)
        x_v = (x_v0, x_v1, x_v2)
        si = (si0, si1, si2)
        sx = (sx0, sx1, sx2)
        so = (so0, so1, so2)

        @pl.when(sid == 0)
        def _load_table():
            pltpu.sync_copy(table2_hbm, table_sh)

        plsc.subcore_barrier()

        def load_idx(q, m):
            pltpu.async_copy(idx2_hbm.at[pl.ds(q, _CH)], idx_va[m], si[m])
            pltpu.async_copy(
                idx2_hbm.at[pl.ds(q + _CH, _CH)], idx_vb[m], si[m])

        def wait_idx(q, m):
            pltpu.make_async_copy(
                idx2_hbm.at[pl.ds(q, _CH)], idx_va[m], si[m]).wait()
            pltpu.make_async_copy(
                idx2_hbm.at[pl.ds(q + _CH, _CH)], idx_vb[m], si[m]).wait()

        for j in range(3):
            q = pbase + j * _CP
            load_idx(q, j)
            pltpu.async_copy(x_hbm.at[pl.ds(q, _CP)], x_v[j], sx[j])

        def chunk(h3, carry):
            for m in range(3):
                h = 3 * h3 + m

                @pl.when(h < _NCH)
                def _do():
                    q0 = pbase + h * _CP
                    wait_idx(q0, m)
                    pltpu.make_async_copy(
                        x_hbm.at[pl.ds(q0, _CP)], x_v[m], sx[m]).wait()
                    pltpu.async_copy(
                        table_sh.at[idx_va[m]],
                        x_v[m].at[pl.ds(0, _CH)], sg, add=True)
                    pltpu.async_copy(
                        table_sh.at[idx_vb[m]],
                        x_v[m].at[pl.ds(_CH, _CH)], sg2, add=True)
                    pltpu.make_async_copy(
                        table_sh.at[idx_va[m]],
                        x_v[m].at[pl.ds(0, _CH)], sg).wait()
                    pltpu.make_async_copy(
                        table_sh.at[idx_vb[m]],
                        x_v[m].at[pl.ds(_CH, _CH)], sg2).wait()
                    pltpu.async_copy(
                        x_v[m], out_hbm.at[pl.ds(q0, _CP)], so[m])

                    @pl.when(h + 3 < _NCH)
                    def _prefetch_idx():
                        load_idx(q0 + 3 * _CP, m)

                    mp = (m + 2) % 3
                    h2 = h + 2

                    @pl.when((h2 >= 3) & (h2 < _NCH))
                    def _prefetch_x():
                        q2 = pbase + h2 * _CP
                        pltpu.make_async_copy(
                            x_v[mp],
                            out_hbm.at[pl.ds(q2 - 3 * _CP, _CP)],
                            so[mp]).wait()
                        pltpu.async_copy(
                            x_hbm.at[pl.ds(q2, _CP)], x_v[mp], sx[mp])

            return carry

        lax.fori_loop(0, (_NCH + 2) // 3, chunk, 0)

        for m in range(3):
            pltpu.make_async_copy(
                x_v[m], out_hbm.at[pl.ds(pbase, _CP)], so[m]).wait()

    pl.run_scoped(
        inner,
        *([pltpu.VMEM((_CH,), jnp.int32)] * 6),
        *([pltpu.VMEM((_CP, _DP), jnp.float32)] * 3),
        *([pltpu.SemaphoreType.DMA] * 11),
    )


_sc_call = pl.kernel(
    _sc_fn,
    mesh=_sc_mesh,
    out_type=jax.ShapeDtypeStruct((_N2, _DP), jnp.float32),
    scratch_types=[pltpu.VMEM_SHARED((1024, _DP), jnp.float32)],
)


def kernel(inputs, x, table):
    B, S, D = x.shape
    idx1 = inputs.reshape(B * S)
    idx2 = idx1[0::2] * 32 + idx1[1::2]
    x2 = x.reshape(_N2, _DP)
    table2 = jnp.concatenate(
        [jnp.repeat(table, 32, axis=0), jnp.tile(table, (32, 1))], axis=1)
    out2 = _sc_call(idx2, x2, table2)
    return out2.reshape(B, S, D)
